# baseline (speedup 1.0000x reference)
import functools

import jax
import jax.numpy as jnp
from jax import lax
from jax.experimental import pallas as pl
from jax.experimental.pallas import tpu as pltpu
from jax.experimental.pallas import tpu_sc as plsc

B = 16384
D = 64
NC = 2
NS = 16
NW = NC * NS
PER_W = B // NW
CHUNK = 128
NCHUNK = PER_W // CHUNK
GROUPS = CHUNK // 16

_MESH = plsc.VectorSubcoreMesh(core_axis_name="c", subcore_axis_name="s")


@functools.partial(
    pl.kernel,
    mesh=_MESH,
    compiler_params=pltpu.CompilerParams(
        needs_layout_passes=False, use_tc_tiling_on_sc=False),
    out_type=[
        jax.ShapeDtypeStruct((B,), jnp.float32),
        jax.ShapeDtypeStruct((B, D), jnp.float32),
        jax.ShapeDtypeStruct((B, D), jnp.float32),
    ],
    scratch_types=[
        pltpu.VMEM((CHUNK,), jnp.int32),
        pltpu.VMEM((CHUNK,), jnp.int32),
        pltpu.VMEM((CHUNK,), jnp.int32),
        pltpu.VMEM((CHUNK, D), jnp.float32),
        pltpu.VMEM((CHUNK, D), jnp.float32),
        pltpu.VMEM((CHUNK, D), jnp.float32),
        pltpu.VMEM((CHUNK, D), jnp.float32),
        pltpu.VMEM((CHUNK, D), jnp.float32),
        pltpu.VMEM((CHUNK, D), jnp.float32),
        pltpu.VMEM((CHUNK, D), jnp.float32),
        pltpu.VMEM((CHUNK, D), jnp.float32),
        pltpu.VMEM((CHUNK,), jnp.float32),
        pltpu.SemaphoreType.DMA,
    ],
)
def _sc_transd(h_hbm, t_hbm, r_hbm, ent_hbm, rel_hbm, entp_hbm, relp_hbm,
               score_hbm, ph_hbm, pt_hbm,
               h_idx, t_idx, r_idx, h_e, t_e, r_e, h_p, t_p, r_p,
               ph_v, pt_v, sc_v, sem):
    wid = lax.axis_index("s") * NC + lax.axis_index("c")
    base = wid * PER_W
    iota16 = lax.iota(jnp.int32, 16)

    def chunk_body(c, carry):
        off = base + c * CHUNK
        pltpu.sync_copy(h_hbm.at[pl.ds(off, CHUNK)], h_idx)
        pltpu.sync_copy(t_hbm.at[pl.ds(off, CHUNK)], t_idx)
        pltpu.sync_copy(r_hbm.at[pl.ds(off, CHUNK)], r_idx)
        copies = [
            pltpu.async_copy(ent_hbm.at[h_idx], h_e, sem),
            pltpu.async_copy(ent_hbm.at[t_idx], t_e, sem),
            pltpu.async_copy(rel_hbm.at[r_idx], r_e, sem),
            pltpu.async_copy(entp_hbm.at[h_idx], h_p, sem),
            pltpu.async_copy(entp_hbm.at[t_idx], t_p, sem),
            pltpu.async_copy(relp_hbm.at[r_idx], r_p, sem),
        ]
        for cp in copies:
            cp.wait()

        def group_body(g, gcarry):
            row = g * 16 + iota16
            sh = jnp.zeros((16,), jnp.float32)
            st = jnp.zeros((16,), jnp.float32)
            for d in range(D):
                col = jnp.full((16,), d, jnp.int32)
                he = plsc.load_gather(h_e, [row, col])
                hp = plsc.load_gather(h_p, [row, col])
                te = plsc.load_gather(t_e, [row, col])
                tp = plsc.load_gather(t_p, [row, col])
                sh = sh + he * hp
                st = st + te * tp
            acc = jnp.zeros((16,), jnp.float32)
            for d in range(D):
                col = jnp.full((16,), d, jnp.int32)
                he = plsc.load_gather(h_e, [row, col])
                te = plsc.load_gather(t_e, [row, col])
                re = plsc.load_gather(r_e, [row, col])
                rp = plsc.load_gather(r_p, [row, col])
                ph = he + sh * rp
                pt = te + st * rp
                plsc.store_scatter(ph_v, [row, col], ph)
                plsc.store_scatter(pt_v, [row, col], pt)
                diff = ph + re - pt
                acc = acc + diff * diff
            sc_v[pl.ds(g * 16, 16)] = acc
            return gcarry

        lax.fori_loop(0, GROUPS, group_body, 0)
        pltpu.sync_copy(ph_v, ph_hbm.at[pl.ds(off, CHUNK), :])
        pltpu.sync_copy(pt_v, pt_hbm.at[pl.ds(off, CHUNK), :])
        pltpu.sync_copy(sc_v, score_hbm.at[pl.ds(off, CHUNK)])
        return carry

    lax.fori_loop(0, NCHUNK, chunk_body, 0)


def kernel(ratings, triples, ent_emb, rel_emb, ent_proj_emb, rel_proj_emb):
    h = triples[0]
    t = triples[1]
    r = triples[2]
    score, proj_h_e, proj_t_e = _sc_transd(
        h, t, r, ent_emb, rel_emb, ent_proj_emb, rel_proj_emb)
    ones = jnp.ones((512, 64), dtype=jnp.float32)
    return (score, proj_h_e, proj_t_e, ones, ones)


# speedup vs baseline: 1.5672x; 1.5672x over previous
import functools

import jax
import jax.numpy as jnp
from jax import lax
from jax.experimental import pallas as pl
from jax.experimental.pallas import tpu as pltpu
from jax.experimental.pallas import tpu_sc as plsc

B = 16384
D = 64
DC = D // 16
NC = 2
NS = 16
NW = NC * NS
PER_W = B // NW
CHUNK = 128
NCHUNK = PER_W // CHUNK
GROUPS = CHUNK // 16
PAD = 17

_MESH = plsc.VectorSubcoreMesh(core_axis_name="c", subcore_axis_name="s")

_BCAST_DNUMS = lax.GatherDimensionNumbers(
    offset_dims=(), collapsed_slice_dims=(0,), start_index_map=(0,))


def _lane_broadcast(vec, j):
    idx = jnp.full((16, 1), j, jnp.int32)
    return lax.gather(vec, idx, _BCAST_DNUMS, (1,),
                      mode=lax.GatherScatterMode.PROMISE_IN_BOUNDS)


@functools.partial(
    pl.kernel,
    mesh=_MESH,
    compiler_params=pltpu.CompilerParams(
        needs_layout_passes=False, use_tc_tiling_on_sc=False),
    out_type=[
        jax.ShapeDtypeStruct((B,), jnp.float32),
        jax.ShapeDtypeStruct((B, D), jnp.float32),
        jax.ShapeDtypeStruct((B, D), jnp.float32),
    ],
    scratch_types=[
        pltpu.VMEM((CHUNK,), jnp.int32),
        pltpu.VMEM((CHUNK,), jnp.int32),
        pltpu.VMEM((CHUNK,), jnp.int32),
        pltpu.VMEM((CHUNK, D), jnp.float32),
        pltpu.VMEM((CHUNK, D), jnp.float32),
        pltpu.VMEM((CHUNK, D), jnp.float32),
        pltpu.VMEM((CHUNK, D), jnp.float32),
        pltpu.VMEM((CHUNK, D), jnp.float32),
        pltpu.VMEM((CHUNK, D), jnp.float32),
        pltpu.VMEM((CHUNK, PAD), jnp.float32),
        pltpu.VMEM((CHUNK, PAD), jnp.float32),
        pltpu.VMEM((CHUNK, PAD), jnp.float32),
        pltpu.VMEM((CHUNK, D), jnp.float32),
        pltpu.VMEM((CHUNK, D), jnp.float32),
        pltpu.VMEM((CHUNK,), jnp.float32),
        pltpu.SemaphoreType.DMA,
    ],
)
def _sc_transd(h_hbm, t_hbm, r_hbm, ent_hbm, rel_hbm, entp_hbm, relp_hbm,
               score_hbm, ph_hbm, pt_hbm,
               h_idx, t_idx, r_idx, h_e, t_e, r_e, h_p, t_p, r_p,
               hd_part, td_part, sc_part,
               ph_v, pt_v, sc_v, sem):
    wid = lax.axis_index("s") * NC + lax.axis_index("c")
    base = wid * PER_W
    iota16 = lax.iota(jnp.int32, 16)

    def chunk_body(c, carry):
        off = base + c * CHUNK
        pltpu.sync_copy(h_hbm.at[pl.ds(off, CHUNK)], h_idx)
        pltpu.sync_copy(t_hbm.at[pl.ds(off, CHUNK)], t_idx)
        pltpu.sync_copy(r_hbm.at[pl.ds(off, CHUNK)], r_idx)
        copies = [
            pltpu.async_copy(ent_hbm.at[h_idx], h_e, sem),
            pltpu.async_copy(ent_hbm.at[t_idx], t_e, sem),
            pltpu.async_copy(rel_hbm.at[r_idx], r_e, sem),
            pltpu.async_copy(entp_hbm.at[h_idx], h_p, sem),
            pltpu.async_copy(entp_hbm.at[t_idx], t_p, sem),
            pltpu.async_copy(relp_hbm.at[r_idx], r_p, sem),
        ]
        for cp in copies:
            cp.wait()

        def dot_body(i, dcarry):
            hd = h_e[i, pl.ds(0, 16)] * h_p[i, pl.ds(0, 16)]
            td = t_e[i, pl.ds(0, 16)] * t_p[i, pl.ds(0, 16)]
            for dc in range(1, DC):
                sl = pl.ds(dc * 16, 16)
                hd = hd + h_e[i, sl] * h_p[i, sl]
                td = td + t_e[i, sl] * t_p[i, sl]
            hd_part[i, pl.ds(0, 16)] = hd
            td_part[i, pl.ds(0, 16)] = td
            return dcarry

        lax.fori_loop(0, CHUNK, dot_body, 0)

        def group_body(g, gcarry):
            row = g * 16 + iota16
            sh = plsc.load_gather(hd_part, [row, jnp.zeros((16,), jnp.int32)])
            st = plsc.load_gather(td_part, [row, jnp.zeros((16,), jnp.int32)])
            for l in range(1, 16):
                col = jnp.full((16,), l, jnp.int32)
                sh = sh + plsc.load_gather(hd_part, [row, col])
                st = st + plsc.load_gather(td_part, [row, col])
            for j in range(16):
                i = g * 16 + j
                shi = _lane_broadcast(sh, j)
                sti = _lane_broadcast(st, j)
                sl0 = pl.ds(0, 16)
                rp = r_p[i, sl0]
                ph = h_e[i, sl0] + shi * rp
                pt = t_e[i, sl0] + sti * rp
                ph_v[i, sl0] = ph
                pt_v[i, sl0] = pt
                diff = ph + r_e[i, sl0] - pt
                acc = diff * diff
                for dc in range(1, DC):
                    sl = pl.ds(dc * 16, 16)
                    rp = r_p[i, sl]
                    ph = h_e[i, sl] + shi * rp
                    pt = t_e[i, sl] + sti * rp
                    ph_v[i, sl] = ph
                    pt_v[i, sl] = pt
                    diff = ph + r_e[i, sl] - pt
                    acc = acc + diff * diff
                sc_part[i, pl.ds(0, 16)] = acc
            sc = plsc.load_gather(sc_part, [row, jnp.zeros((16,), jnp.int32)])
            for l in range(1, 16):
                col = jnp.full((16,), l, jnp.int32)
                sc = sc + plsc.load_gather(sc_part, [row, col])
            sc_v[pl.ds(g * 16, 16)] = sc
            return gcarry

        lax.fori_loop(0, GROUPS, group_body, 0)
        pltpu.sync_copy(ph_v, ph_hbm.at[pl.ds(off, CHUNK), :])
        pltpu.sync_copy(pt_v, pt_hbm.at[pl.ds(off, CHUNK), :])
        pltpu.sync_copy(sc_v, score_hbm.at[pl.ds(off, CHUNK)])
        return carry

    lax.fori_loop(0, NCHUNK, chunk_body, 0)


def kernel(ratings, triples, ent_emb, rel_emb, ent_proj_emb, rel_proj_emb):
    h = triples[0]
    t = triples[1]
    r = triples[2]
    score, proj_h_e, proj_t_e = _sc_transd(
        h, t, r, ent_emb, rel_emb, ent_proj_emb, rel_proj_emb)
    ones = jnp.ones((512, 64), dtype=jnp.float32)
    return (score, proj_h_e, proj_t_e, ones, ones)


# speedup vs baseline: 1.7262x; 1.1014x over previous
import functools

import jax
import jax.numpy as jnp
from jax import lax
from jax.experimental import pallas as pl
from jax.experimental.pallas import tpu as pltpu
from jax.experimental.pallas import tpu_sc as plsc

B = 16384
D = 64
DC = D // 16
NC = 2
NS = 16
NW = NC * NS
PER_W = B // NW
CHUNK = 128
NCHUNK = PER_W // CHUNK
GROUPS = CHUNK // 16
PAD = 17

_MESH = plsc.VectorSubcoreMesh(core_axis_name="c", subcore_axis_name="s")

_BCAST_DNUMS = lax.GatherDimensionNumbers(
    offset_dims=(), collapsed_slice_dims=(0,), start_index_map=(0,))


def _lane_broadcast(vec, j):
    idx = jnp.full((16, 1), j, jnp.int32)
    return lax.gather(vec, idx, _BCAST_DNUMS, (1,),
                      mode=lax.GatherScatterMode.PROMISE_IN_BOUNDS)


@functools.partial(
    pl.kernel,
    mesh=_MESH,
    compiler_params=pltpu.CompilerParams(
        needs_layout_passes=False, use_tc_tiling_on_sc=True),
    out_type=[
        jax.ShapeDtypeStruct((B,), jnp.float32),
        jax.ShapeDtypeStruct((B * D,), jnp.float32),
        jax.ShapeDtypeStruct((B * D,), jnp.float32),
    ],
    scratch_types=[
        pltpu.VMEM((CHUNK,), jnp.int32),
        pltpu.VMEM((CHUNK,), jnp.int32),
        pltpu.VMEM((CHUNK,), jnp.int32),
        pltpu.VMEM((CHUNK, 2 * D), jnp.float32),
        pltpu.VMEM((CHUNK, 2 * D), jnp.float32),
        pltpu.VMEM((CHUNK, 2 * D), jnp.float32),
        pltpu.VMEM((CHUNK * PAD,), jnp.float32),
        pltpu.VMEM((CHUNK * PAD,), jnp.float32),
        pltpu.VMEM((CHUNK * PAD,), jnp.float32),
        pltpu.VMEM((CHUNK * D,), jnp.float32),
        pltpu.VMEM((CHUNK * D,), jnp.float32),
        pltpu.VMEM((CHUNK,), jnp.float32),
        pltpu.SemaphoreType.DMA,
    ],
)
def _sc_transd(h_hbm, t_hbm, r_hbm, entc_hbm, relc_hbm,
               score_hbm, ph_hbm, pt_hbm,
               h_idx, t_idx, r_idx, h_b, t_b, r_b,
               hd_part, td_part, sc_part,
               ph_v, pt_v, sc_v, sem):
    wid = lax.axis_index("s") * NC + lax.axis_index("c")
    base = wid * PER_W
    iota16 = lax.iota(jnp.int32, 16)
    row17 = iota16 * PAD

    def chunk_body(c, carry):
        off = base + c * CHUNK
        pltpu.sync_copy(h_hbm.at[pl.ds(off, CHUNK)], h_idx)
        pltpu.sync_copy(t_hbm.at[pl.ds(off, CHUNK)], t_idx)
        pltpu.sync_copy(r_hbm.at[pl.ds(off, CHUNK)], r_idx)
        copies = [
            pltpu.async_copy(entc_hbm.at[h_idx], h_b, sem),
            pltpu.async_copy(entc_hbm.at[t_idx], t_b, sem),
            pltpu.async_copy(relc_hbm.at[r_idx], r_b, sem),
        ]
        for cp in copies:
            cp.wait()

        def dot_body(i, dcarry):
            hd = h_b[i, pl.ds(0, 16)] * h_b[i, pl.ds(D, 16)]
            td = t_b[i, pl.ds(0, 16)] * t_b[i, pl.ds(D, 16)]
            for dc in range(1, DC):
                sl = pl.ds(dc * 16, 16)
                slp = pl.ds(D + dc * 16, 16)
                hd = hd + h_b[i, sl] * h_b[i, slp]
                td = td + t_b[i, sl] * t_b[i, slp]
            hd_part[pl.ds(i * PAD, 16)] = hd
            td_part[pl.ds(i * PAD, 16)] = td
            return dcarry

        lax.fori_loop(0, CHUNK, dot_body, 0)

        def group_body(g, gcarry):
            grow = g * (16 * PAD) + row17
            sh = plsc.load_gather(hd_part, [grow])
            st = plsc.load_gather(td_part, [grow])
            for l in range(1, 16):
                gl = grow + l
                sh = sh + plsc.load_gather(hd_part, [gl])
                st = st + plsc.load_gather(td_part, [gl])
            for j in range(16):
                i = g * 16 + j
                shi = _lane_broadcast(sh, j)
                sti = _lane_broadcast(st, j)
                sl0 = pl.ds(0, 16)
                slp0 = pl.ds(D, 16)
                rp = r_b[i, slp0]
                ph = h_b[i, sl0] + shi * rp
                pt = t_b[i, sl0] + sti * rp
                ph_v[pl.ds(i * D, 16)] = ph
                pt_v[pl.ds(i * D, 16)] = pt
                diff = ph + r_b[i, sl0] - pt
                acc = diff * diff
                for dc in range(1, DC):
                    sl = pl.ds(dc * 16, 16)
                    slp = pl.ds(D + dc * 16, 16)
                    rp = r_b[i, slp]
                    ph = h_b[i, sl] + shi * rp
                    pt = t_b[i, sl] + sti * rp
                    ph_v[pl.ds(i * D + dc * 16, 16)] = ph
                    pt_v[pl.ds(i * D + dc * 16, 16)] = pt
                    diff = ph + r_b[i, sl] - pt
                    acc = acc + diff * diff
                sc_part[pl.ds(i * PAD, 16)] = acc
            sc = plsc.load_gather(sc_part, [grow])
            for l in range(1, 16):
                sc = sc + plsc.load_gather(sc_part, [grow + l])
            sc_v[pl.ds(g * 16, 16)] = sc
            return gcarry

        lax.fori_loop(0, GROUPS, group_body, 0)
        pltpu.sync_copy(ph_v, ph_hbm.at[pl.ds(off * D, CHUNK * D)])
        pltpu.sync_copy(pt_v, pt_hbm.at[pl.ds(off * D, CHUNK * D)])
        pltpu.sync_copy(sc_v, score_hbm.at[pl.ds(off, CHUNK)])
        return carry

    lax.fori_loop(0, NCHUNK, chunk_body, 0)


def kernel(ratings, triples, ent_emb, rel_emb, ent_proj_emb, rel_proj_emb):
    h = triples[0]
    t = triples[1]
    r = triples[2]
    entc = jnp.concatenate([ent_emb, ent_proj_emb], axis=1)
    relc = jnp.concatenate([rel_emb, rel_proj_emb], axis=1)
    score, ph_flat, pt_flat = _sc_transd(h, t, r, entc, relc)
    proj_h_e = ph_flat.reshape(B, D)
    proj_t_e = pt_flat.reshape(B, D)
    ones = jnp.ones((512, 64), dtype=jnp.float32)
    return (score, proj_h_e, proj_t_e, ones, ones)
